# baseline (speedup 1.0000x reference)
STAGE = 6
import functools
import jax
import jax.numpy as jnp
from jax import lax
from jax.experimental import pallas as pl
from jax.experimental.pallas import tpu as pltpu
from jax.experimental.pallas import tpu_sc as plsc

N = 8
E = 32

X, EW, NZ, W1O, B1O, W2O, B2O = 0, 64, 96, 112, 624, 688, 752
FC1O, F1BO, FC2O, F2BO, FC3O, F3BO = 768, 1536, 1600, 5696, 5760, 5824
FTOT = 5840

DINV, Q, G, COMB, Z, H = 0, 16, 32, 48, 80, 592
WTOT = 1104


def _bf16r(v):
    i = lax.bitcast_convert_type(v, jnp.int32)
    lsb = lax.shift_right_logical(i, 16) & 1
    r = (i + 0x7FFF + lsb) & jnp.int32(-65536)
    return lax.bitcast_convert_type(r, jnp.float32)


def _bits(v):
    return lax.bitcast_convert_type(v, jnp.int32)


def _flt(i):
    return lax.bitcast_convert_type(i, jnp.float32)


def _cr_sqrt(x):
    i = _bits(x)
    y = _flt(jnp.int32(0x5F3759DF) - lax.shift_right_logical(i, 1))
    for _ in range(4):
        y = y * (1.5 - 0.5 * x * y * y)
    s = x * y
    si = _bits(s)
    best = None
    best_err = None
    for delta in (-2, -1, 0, 1, 2):
        c = _flt(si + delta)
        p = c * c
        t = c * 4097.0
        chi = t - (t - c)
        clo = c - chi
        e = ((chi * chi - p) + 2.0 * (chi * clo)) + clo * clo
        err = jnp.abs((p - x) + e)
        if best is None:
            best, best_err = c, err
        else:
            take = err < best_err
            best = jnp.where(take, c, best)
            best_err = jnp.where(take, err, best_err)
    return best


def _sc_body(fh, eh, oh, fb, eb, w, ov):
    cid = lax.axis_index("c")
    sid = lax.axis_index("s")

    @pl.when(jnp.logical_and(cid == 0, sid == 0))
    def _():
        pltpu.sync_copy(fh, fb)
        pltpu.sync_copy(eh, eb)
        iot = lax.broadcasted_iota(jnp.int32, (16,), 0)
        zeros = jnp.zeros((16,), jnp.float32)

        srcv = [eb[pl.ds(0, 16)], eb[pl.ds(16, 16)]]
        dstv = [eb[pl.ds(E, 16)], eb[pl.ds(E + 16, 16)]]
        eww = [fb[pl.ds(EW, 16)], fb[pl.ds(EW + 16, 16)]]

        degv = zeros
        for c in range(2):
            for j in range(16):
                degv = degv + jnp.where(iot == dstv[c][j], eww[c][j], 0.0)
        degv = degv + jnp.where(iot < N, 1.0, 0.0)

        degv = jnp.where(iot < N, degv, 1.0)
        dinv = 1.0 / _cr_sqrt(degv)
        w[pl.ds(DINV, 16)] = dinv

        def sel8(idxv, vals):
            r = vals[7]
            for n in range(6, -1, -1):
                r = jnp.where(idxv == n, vals[n], r)
            return r

        dlanes = [dinv[n] for n in range(N)]
        normv = []
        for c in range(2):
            dvs = sel8(srcv[c], dlanes)
            dvd = sel8(dstv[c], dlanes)
            normv.append((dvs * eww[c]) * dvd)

        xr = [fb[pl.ds(X + c * 16, 16)] for c in range(4)]

        w1r = [[fb[pl.ds(W1O + k * 64 + c * 16, 16)] for c in range(4)]
               for k in range(N)]
        for i in range(N):
            accs = [None] * 4
            for k in range(N):
                fi = i * N + k
                xs = xr[fi // 16][fi % 16]
                for c in range(4):
                    p = xs * w1r[k][c]
                    accs[c] = p if k == 0 else accs[c] + p
            for c in range(4):
                w[pl.ds(Z + i * 64 + c * 16, 16)] = accs[c]

        Av = {}
        for s2 in range(N):
            Ms = [jnp.where(srcv[c] == s2, normv[c], 0.0) for c in range(2)]
            for d in range(N):
                acc = jnp.float32(0.0)
                for c in range(2):
                    t = jnp.where(dstv[c] == d, Ms[c], 0.0)
                    for j in range(16):
                        acc = acc + t[j]
                Av[(d, s2)] = acc
        for n in range(N):
            Av[(n, n)] = Av[(n, n)] + (dinv[n] * 1.0) * dinv[n]

        for c in range(4):
            zr = [w[pl.ds(Z + i * 64 + c * 16, 16)] for i in range(N)]
            b1c = fb[pl.ds(B1O + c * 16, 16)]
            for d in range(N):
                acc = Av[(d, 0)] * zr[0]
                for s2 in range(1, N):
                    acc = acc + Av[(d, s2)] * zr[s2]
                w[pl.ds(H + d * 64 + c * 16, 16)] = jnp.maximum(acc + b1c, 0.0)

        w2r = [fb[pl.ds(W2O + c * 16, 16)] for c in range(4)]
        qv = zeros
        for i in range(N):
            carry = jnp.float32(0.0)
            for c in range(4):
                hr = _bf16r(w[pl.ds(H + i * 64 + c * 16, 16)])
                prod = hr * w2r[c]
                for j in range(16):
                    carry = carry + prod[j]
            qv = jnp.where(iot == i, carry, qv)
        w[pl.ds(Q, 16)] = qv

        qlanes = [qv[n] for n in range(N)]
        gv = zeros
        for d in range(N):
            gs = Av[(d, 0)] * qlanes[0]
            for s2 in range(1, N):
                gs = gs + Av[(d, s2)] * qlanes[s2]
            gv = jnp.where(iot == d, gs, gv)

        gv = gv + fb[pl.ds(B2O, 16)][0]
        w[pl.ds(COMB, 16)] = gv
        w[pl.ds(COMB + 8, 16)] = fb[pl.ds(NZ, 16)]
        combr = _bf16r(w[pl.ds(COMB, 16)])

        accs = [None] * 4
        for k in range(12):
            cs = combr[k]
            for c in range(4):
                p = cs * fb[pl.ds(FC1O + k * 64 + c * 16, 16)]
                accs[c] = p if k == 0 else accs[c] + p
        h1r = [_bf16r(jnp.maximum(accs[c] + fb[pl.ds(F1BO + c * 16, 16)], 0.0))
               for c in range(4)]

        accs = [None] * 4
        for k in range(64):
            hs = h1r[k // 16][k % 16]
            for c in range(4):
                p = hs * fb[pl.ds(FC2O + k * 64 + c * 16, 16)]
                accs[c] = p if k == 0 else accs[c] + p
        h2 = [jnp.maximum(accs[c] + fb[pl.ds(F2BO + c * 16, 16)], 0.0)
              for c in range(4)]

        acc = h2[0] * fb[pl.ds(FC3O, 16)]
        for c in range(1, 4):
            acc = acc + h2[c] * fb[pl.ds(FC3O + c * 16, 16)]
        out_s = jnp.float32(0.0)
        for j in range(16):
            out_s = out_s + acc[j]
        out_s = out_s + fb[pl.ds(F3BO, 16)][0]
        ov[...] = jnp.where(iot == 0, out_s, 0.0)
        pltpu.sync_copy(ov, oh)


@jax.jit
def _sc_call(fbuf, ebuf):
    mesh = plsc.VectorSubcoreMesh(core_axis_name="c", subcore_axis_name="s", num_cores=1)
    k = functools.partial(
        pl.kernel,
        out_type=jax.ShapeDtypeStruct((16,), jnp.float32),
        mesh=mesh,
        scratch_types=[
            pltpu.VMEM((FTOT,), jnp.float32),
            pltpu.VMEM((64,), jnp.int32),
            pltpu.VMEM((WTOT,), jnp.float32),
            pltpu.VMEM((16,), jnp.float32),
        ],
    )(_sc_body)
    return k(fbuf, ebuf)


def kernel(x, edge_index, edge_weight, noisy_value, W1, b1, W2, b2,
           fc1_W, fc1_b, fc2_W, fc2_b, fc3_W, fc3_b):
    f32 = jnp.float32
    z12 = jnp.zeros((12,), f32)
    z15 = jnp.zeros((15,), f32)
    noisy = jax.lax.stop_gradient(noisy_value).reshape(-1)

    def bfr(a):
        i = lax.bitcast_convert_type(a, jnp.int32)
        lsb = lax.shift_right_logical(i, 16) & 1
        r = (i + 0x7FFF + lsb) & jnp.int32(-65536)
        return lax.bitcast_convert_type(r, f32)

    fbuf = jnp.concatenate([
        bfr(x).reshape(-1), edge_weight, noisy, z12,
        bfr(W1).reshape(-1), b1, bfr(W2).reshape(-1), b2, z15,
        bfr(fc1_W).reshape(-1), fc1_b, bfr(fc2_W).reshape(-1), fc2_b,
        fc3_W.reshape(-1), fc3_b, z15,
    ]).astype(f32)
    ebuf = edge_index.reshape(-1).astype(jnp.int32)
    out = _sc_call(fbuf, ebuf)
    return out[:1].reshape(1, 1)


# speedup vs baseline: 7.0301x; 7.0301x over previous
import jax
import jax.numpy as jnp
from jax.experimental import pallas as pl

N = 8
E = 32


def _bf(a):
    return a.astype(jnp.bfloat16)


def _dot(a, b):
    return jnp.dot(_bf(a), _bf(b), preferred_element_type=jnp.float32)


def _dot_exact(a, b):
    return jnp.dot(a, b, preferred_element_type=jnp.float32,
                   precision=jax.lax.Precision.HIGHEST)


def _fused_kernel(edge_ref, ew_ref, x_ref, noisy_ref, W1_ref, b1_ref,
                  W2_ref, b2_ref, fc1_ref, fc1b_bias_ref,
                  fc2_ref, fc2b_ref, fc3_ref, fc3b_ref, out_ref):
    src_row = edge_ref[0:1, :]
    dst_row = edge_ref[1:2, :]
    ew_row = ew_ref[...]

    iota_d = jax.lax.broadcasted_iota(jnp.int32, (N, E), 0)
    M = jnp.where(iota_d == dst_row, ew_row, 0.0)

    deg_col = jnp.sum(M, axis=1, keepdims=True) + 1.0
    dinv_col = jnp.where(deg_col > 0, 1.0 / jnp.sqrt(deg_col), 0.0)

    cols = []
    deg_row_parts = []
    for s in range(N):
        mask_s = (src_row == s)
        cols.append(jnp.sum(jnp.where(mask_s, M, 0.0), axis=1, keepdims=True))
        deg_row_parts.append(
            jnp.sum(jnp.where(dst_row == s, ew_row, 0.0), axis=1,
                    keepdims=True))
    A = jnp.concatenate(cols, axis=1)
    deg_row = jnp.concatenate(deg_row_parts, axis=1) + 1.0
    dinv_row = jnp.where(deg_row > 0, 1.0 / jnp.sqrt(deg_row), 0.0)

    eye = (jax.lax.broadcasted_iota(jnp.int32, (N, N), 0) ==
           jax.lax.broadcasted_iota(jnp.int32, (N, N), 1)).astype(jnp.float32)
    A_hat = dinv_col * (A + eye) * dinv_row

    Z = _dot(x_ref[...], W1_ref[...])
    H = jnp.maximum(_dot_exact(A_hat, Z) + b1_ref[...], 0.0)

    q = _dot(H, W2_ref[...])
    g = _dot_exact(A_hat, q) + b2_ref[...]

    gT = jnp.sum(jnp.where(eye > 0, g, 0.0), axis=0, keepdims=True)
    combined = jnp.concatenate([gT, noisy_ref[...]], axis=1)
    h1 = jnp.maximum(_dot(combined, fc1_ref[...]) + fc1b_bias_ref[...], 0.0)

    h2 = jnp.maximum(_dot(h1, fc2_ref[...]) + fc2b_ref[...], 0.0)

    out_ref[...] = (jnp.sum(h2 * fc3_ref[...], axis=1, keepdims=True)
                    + fc3b_ref[...])


def kernel(x, edge_index, edge_weight, noisy_value, W1, b1, W2, b2,
           fc1_W, fc1_b, fc2_W, fc2_b, fc3_W, fc3_b):
    ew_row = edge_weight.reshape(1, E)
    noisy = jax.lax.stop_gradient(noisy_value)
    args = (
        edge_index, ew_row, x, noisy,
        W1, b1.reshape(1, -1), W2, b2.reshape(1, -1),
        fc1_W, fc1_b.reshape(1, -1),
        fc2_W, fc2_b.reshape(1, -1), fc3_W.reshape(1, -1),
        fc3_b.reshape(1, -1),
    )
    return pl.pallas_call(
        _fused_kernel,
        out_shape=jax.ShapeDtypeStruct((1, 1), jnp.float32),
    )(*args)
